# Initial kernel scaffold; baseline (speedup 1.0000x reference)
import jax
import jax.numpy as jnp
from jax.experimental import pallas as pl


def kernel(x, edge_index, batch, params):
    raise NotImplementedError("write your pallas kernel here")


# speedup vs baseline: 7.2140x; 7.2140x over previous
import functools

import jax
import jax.numpy as jnp
from jax import lax
from jax.experimental import pallas as pl
from jax.experimental.pallas import tpu as pltpu
from jax.experimental.pallas import tpu_sc as plsc

N = 10000
E = 320000
H = 128
C = 10
L = 3
G = 128
BN_EPS = 1e-5

NC = 2
NS = 16
NW = NC * NS

EB = 128
K_TILE = 80
E_PAD = NW * K_TILE * EB

N_PAD = 10240
RPT = N_PAD // NS
BLK = 1024
NBLK = N_PAD // BLK


def _sc_agg(h_pad, src2d, dst2d):
    mesh = plsc.VectorSubcoreMesh(
        core_axis_name="c", subcore_axis_name="s", num_cores=NC, num_subcores=NS
    )

    @functools.partial(
        pl.kernel,
        out_type=jax.ShapeDtypeStruct((NC, N_PAD, H), jnp.float32),
        mesh=mesh,
        scratch_types=[
            pltpu.VMEM((K_TILE, EB), jnp.int32),
            pltpu.VMEM((K_TILE, EB), jnp.int32),
            pltpu.VMEM((EB, H), jnp.float32),
            pltpu.VMEM_SHARED((N_PAD, H), jnp.float32),
            pltpu.SemaphoreType.DMA,
        ],
    )
    def agg(h_hbm, src_hbm, dst_hbm, out_hbm, src_v, dst_v, rows_v, acc, sem):
        c = lax.axis_index("c")
        s = lax.axis_index("s")
        wid = s * NC + c
        pltpu.sync_copy(h_hbm.at[pl.ds(s * RPT, RPT)], acc.at[pl.ds(s * RPT, RPT)])
        base = wid * K_TILE
        pltpu.sync_copy(src_hbm.at[pl.ds(base, K_TILE)], src_v)
        pltpu.sync_copy(dst_hbm.at[pl.ds(base, K_TILE)], dst_v)
        plsc.subcore_barrier()

        def body(j, carry):
            pltpu.async_copy(h_hbm.at[src_v.at[j]], rows_v, sem).wait()
            pltpu.sync_copy(rows_v, acc.at[dst_v.at[j]], add=True)
            return carry

        lax.fori_loop(0, K_TILE, body, 0)
        plsc.subcore_barrier()
        pltpu.sync_copy(
            acc.at[pl.ds(s * RPT, RPT)], out_hbm.at[c, pl.ds(s * RPT, RPT)]
        )

    return agg(h_pad, src2d, dst2d)


def _mlp_body(h_ref, a_ref, b_ref, w1_ref, bb_ref, w2_ref, b2_ref, o_ref):
    z = a_ref[...] + b_ref[...] - h_ref[...]
    t = jnp.dot(z, w1_ref[...], preferred_element_type=jnp.float32) + bb_ref[0:1, :]
    t = jnp.maximum(t, 0.0)
    u = jnp.dot(t, w2_ref[...], preferred_element_type=jnp.float32) + b2_ref[0:1, :]
    o_ref[...] = jnp.maximum(u, 0.0)


def _mlp(h_pad, acc_a, acc_b, w1f, bb, w2, b2):
    row = lambda i: (i, 0)
    fixed = lambda i: (0, 0)
    return pl.pallas_call(
        _mlp_body,
        grid=(NBLK,),
        in_specs=[
            pl.BlockSpec((BLK, H), row),
            pl.BlockSpec((BLK, H), row),
            pl.BlockSpec((BLK, H), row),
            pl.BlockSpec((H, H), fixed),
            pl.BlockSpec((8, H), fixed),
            pl.BlockSpec((H, H), fixed),
            pl.BlockSpec((8, H), fixed),
        ],
        out_specs=pl.BlockSpec((BLK, H), row),
        out_shape=jax.ShapeDtypeStruct((N_PAD, H), jnp.float32),
    )(h_pad, acc_a, acc_b, w1f, bb, w2, b2)


def _pool_body(h_ref, batch_ref, wout_ref, bout_ref, o_ref, pool_acc):
    i = pl.program_id(0)
    bi = batch_ref[0, 0, :]
    seg = lax.broadcasted_iota(jnp.int32, (G, BLK), 0)
    m = (bi[None, :] == seg).astype(jnp.float32)
    part = jnp.dot(m, h_ref[...], preferred_element_type=jnp.float32)

    @pl.when(i == 0)
    def _():
        pool_acc[...] = jnp.zeros_like(pool_acc)

    pool_acc[...] += part

    @pl.when(i == NBLK - 1)
    def _():
        o_ref[...] = (
            jnp.dot(pool_acc[...], wout_ref[...], preferred_element_type=jnp.float32)
            + bout_ref[0:1, :]
        )


def _pool(h_pad, batch3d, wout_p, bout_p):
    return pl.pallas_call(
        _pool_body,
        grid=(NBLK,),
        in_specs=[
            pl.BlockSpec((BLK, H), lambda i: (i, 0)),
            pl.BlockSpec((1, 1, BLK), lambda i: (i, 0, 0)),
            pl.BlockSpec((H, H), lambda i: (0, 0)),
            pl.BlockSpec((8, H), lambda i: (0, 0)),
        ],
        out_specs=pl.BlockSpec((G, H), lambda i: (0, 0)),
        out_shape=jax.ShapeDtypeStruct((G, H), jnp.float32),
        scratch_shapes=[pltpu.VMEM((G, H), jnp.float32)],
    )(h_pad, batch3d, wout_p, bout_p)


def kernel(x, edge_index, batch, params):
    h = jnp.zeros((N_PAD, H), jnp.float32).at[:N].set(x)

    pad_n = E_PAD - E
    fill = (N + (jnp.arange(pad_n, dtype=jnp.int32) % (N_PAD - N))).astype(jnp.int32)
    src2d = jnp.concatenate([edge_index[0], fill]).reshape(NW * K_TILE, EB)
    dst2d = jnp.concatenate([edge_index[1], fill]).reshape(NW * K_TILE, EB)

    inv_std = 1.0 / jnp.sqrt(1.0 + BN_EPS)
    bc8 = lambda v: jnp.broadcast_to(v[None, :], (8, H))

    for i in range(L):
        sc = inv_std * params[f"gamma_{i}"]
        w1f = params[f"W1_{i}"] * sc[None, :]
        bb = bc8(params[f"b1_{i}"] * sc + params[f"beta_{i}"])
        b2 = bc8(params[f"b2_{i}"])
        acc = _sc_agg(h, src2d, dst2d)
        h = _mlp(h, acc[0], acc[1], w1f, bb, params[f"W2_{i}"], b2)

    batch3d = (
        jnp.concatenate([batch, jnp.full((N_PAD - N,), G, jnp.int32)])
        .reshape(NBLK, 1, BLK)
    )
    wout_p = jnp.zeros((H, H), jnp.float32).at[:, :C].set(params["Wout"])
    bout_p = bc8(jnp.zeros((H,), jnp.float32).at[:C].set(params["bout"]))
    out = _pool(h, batch3d, wout_p, bout_p)
    return out[:, :C]


# speedup vs baseline: 10.6762x; 1.4799x over previous
import functools

import jax
import jax.numpy as jnp
from jax import lax
from jax.experimental import pallas as pl
from jax.experimental.pallas import tpu as pltpu
from jax.experimental.pallas import tpu_sc as plsc

N = 10000
E = 320000
H = 128
C = 10
L = 3
G = 128
BN_EPS = 1e-5

NC = 2
NS = 16
NW = NC * NS

EB = 128
K_TILE = 80
E_PAD = NW * K_TILE * EB
GRP = 8
NGRP = K_TILE // GRP

N_PAD = 10240
RPT = N_PAD // NS
BLK = 1024
NBLK = N_PAD // BLK


def _sc_agg(h_pad, src2d, dst2d):
    mesh = plsc.VectorSubcoreMesh(
        core_axis_name="c", subcore_axis_name="s", num_cores=NC, num_subcores=NS
    )

    @functools.partial(
        pl.kernel,
        out_type=jax.ShapeDtypeStruct((NC, N_PAD, H), jnp.float32),
        mesh=mesh,
        scratch_types=[
            pltpu.VMEM((K_TILE, EB), jnp.int32),
            pltpu.VMEM((2, GRP, EB), jnp.int32),
            pltpu.VMEM((EB, H), jnp.float32),
            pltpu.VMEM((EB, H), jnp.float32),
            pltpu.VMEM_SHARED((N_PAD, H), jnp.float32),
            pltpu.SemaphoreType.DMA,
            pltpu.SemaphoreType.DMA,
            pltpu.SemaphoreType.DMA,
        ],
    )
    def agg(h_hbm, src_hbm, dst_hbm, out_hbm, src_v, dst_i, rows_a, rows_b, acc, sem_a, sem_b, sem_i):
        c = lax.axis_index("c")
        s = lax.axis_index("s")
        wid = s * NC + c
        pltpu.sync_copy(h_hbm.at[pl.ds(s * RPT, RPT)], acc.at[pl.ds(s * RPT, RPT)])
        base = wid * K_TILE
        pltpu.sync_copy(src_hbm.at[pl.ds(base, K_TILE)], src_v)
        pltpu.sync_copy(dst_hbm.at[pl.ds(base, GRP)], dst_i.at[0])
        plsc.subcore_barrier()

        pltpu.async_copy(h_hbm.at[src_v.at[0]], rows_a, sem_a)

        def group(g, carry):
            slot = lax.rem(g, 2)

            @pl.when(g + 1 < NGRP)
            def _():
                pltpu.async_copy(
                    dst_hbm.at[pl.ds(base + (g + 1) * GRP, GRP)],
                    dst_i.at[lax.rem(g + 1, 2)],
                    sem_i,
                )

            for p in range(GRP // 2):
                j0 = g * GRP + 2 * p
                pltpu.async_copy(h_hbm.at[src_v.at[j0 + 1]], rows_b, sem_b)
                pltpu.make_async_copy(h_hbm.at[src_v.at[j0]], rows_a, sem_a).wait()
                pltpu.sync_copy(rows_a, acc.at[dst_i.at[slot, 2 * p]], add=True)
                if p + 1 < GRP // 2:
                    pltpu.async_copy(h_hbm.at[src_v.at[j0 + 2]], rows_a, sem_a)
                else:

                    @pl.when(g + 1 < NGRP)
                    def _():
                        pltpu.async_copy(h_hbm.at[src_v.at[j0 + 2]], rows_a, sem_a)

                pltpu.make_async_copy(h_hbm.at[src_v.at[j0 + 1]], rows_b, sem_b).wait()
                pltpu.sync_copy(rows_b, acc.at[dst_i.at[slot, 2 * p + 1]], add=True)

            @pl.when(g + 1 < NGRP)
            def _():
                pltpu.make_async_copy(
                    dst_hbm.at[pl.ds(base + (g + 1) * GRP, GRP)],
                    dst_i.at[lax.rem(g + 1, 2)],
                    sem_i,
                ).wait()

            return carry

        lax.fori_loop(0, NGRP, group, 0)
        plsc.subcore_barrier()
        pltpu.sync_copy(
            acc.at[pl.ds(s * RPT, RPT)], out_hbm.at[c, pl.ds(s * RPT, RPT)]
        )

    return agg(h_pad, src2d, dst2d)


def _mlp_body(h_ref, a_ref, b_ref, w1_ref, bb_ref, w2_ref, b2_ref, o_ref):
    z = a_ref[...] + b_ref[...] - h_ref[...]
    t = jnp.dot(z, w1_ref[...], preferred_element_type=jnp.float32) + bb_ref[0:1, :]
    t = jnp.maximum(t, 0.0)
    u = jnp.dot(t, w2_ref[...], preferred_element_type=jnp.float32) + b2_ref[0:1, :]
    o_ref[...] = jnp.maximum(u, 0.0)


def _mlp(h_pad, acc_a, acc_b, w1f, bb, w2, b2):
    row = lambda i: (i, 0)
    fixed = lambda i: (0, 0)
    return pl.pallas_call(
        _mlp_body,
        grid=(NBLK,),
        in_specs=[
            pl.BlockSpec((BLK, H), row),
            pl.BlockSpec((BLK, H), row),
            pl.BlockSpec((BLK, H), row),
            pl.BlockSpec((H, H), fixed),
            pl.BlockSpec((8, H), fixed),
            pl.BlockSpec((H, H), fixed),
            pl.BlockSpec((8, H), fixed),
        ],
        out_specs=pl.BlockSpec((BLK, H), row),
        out_shape=jax.ShapeDtypeStruct((N_PAD, H), jnp.float32),
    )(h_pad, acc_a, acc_b, w1f, bb, w2, b2)


def _pool_body(h_ref, batch_ref, wout_ref, bout_ref, o_ref, pool_acc):
    i = pl.program_id(0)
    bi = batch_ref[0, 0, :]
    seg = lax.broadcasted_iota(jnp.int32, (G, BLK), 0)
    m = (bi[None, :] == seg).astype(jnp.float32)
    part = jnp.dot(m, h_ref[...], preferred_element_type=jnp.float32)

    @pl.when(i == 0)
    def _():
        pool_acc[...] = jnp.zeros_like(pool_acc)

    pool_acc[...] += part

    @pl.when(i == NBLK - 1)
    def _():
        o_ref[...] = (
            jnp.dot(pool_acc[...], wout_ref[...], preferred_element_type=jnp.float32)
            + bout_ref[0:1, :]
        )


def _pool(h_pad, batch3d, wout_p, bout_p):
    return pl.pallas_call(
        _pool_body,
        grid=(NBLK,),
        in_specs=[
            pl.BlockSpec((BLK, H), lambda i: (i, 0)),
            pl.BlockSpec((1, 1, BLK), lambda i: (i, 0, 0)),
            pl.BlockSpec((H, H), lambda i: (0, 0)),
            pl.BlockSpec((8, H), lambda i: (0, 0)),
        ],
        out_specs=pl.BlockSpec((G, H), lambda i: (0, 0)),
        out_shape=jax.ShapeDtypeStruct((G, H), jnp.float32),
        scratch_shapes=[pltpu.VMEM((G, H), jnp.float32)],
    )(h_pad, batch3d, wout_p, bout_p)


def kernel(x, edge_index, batch, params):
    h = jnp.zeros((N_PAD, H), jnp.float32).at[:N].set(x)

    pad_n = E_PAD - E
    fill = (N + (jnp.arange(pad_n, dtype=jnp.int32) % (N_PAD - N))).astype(jnp.int32)
    src2d = jnp.concatenate([edge_index[0], fill]).reshape(NW * K_TILE, EB)
    dst2d = jnp.concatenate([edge_index[1], fill]).reshape(NW * K_TILE, EB)

    inv_std = 1.0 / jnp.sqrt(1.0 + BN_EPS)
    bc8 = lambda v: jnp.broadcast_to(v[None, :], (8, H))

    for i in range(L):
        sc = inv_std * params[f"gamma_{i}"]
        w1f = params[f"W1_{i}"] * sc[None, :]
        bb = bc8(params[f"b1_{i}"] * sc + params[f"beta_{i}"])
        b2 = bc8(params[f"b2_{i}"])
        acc = _sc_agg(h, src2d, dst2d)
        h = _mlp(h, acc[0], acc[1], w1f, bb, params[f"W2_{i}"], b2)

    batch3d = (
        jnp.concatenate([batch, jnp.full((N_PAD - N,), G, jnp.int32)])
        .reshape(NBLK, 1, BLK)
    )
    wout_p = jnp.zeros((H, H), jnp.float32).at[:, :C].set(params["Wout"])
    bout_p = bc8(jnp.zeros((H,), jnp.float32).at[:C].set(params["bout"]))
    out = _pool(h, batch3d, wout_p, bout_p)
    return out[:, :C]


# speedup vs baseline: 11.9836x; 1.1225x over previous
import functools

import jax
import jax.numpy as jnp
from jax import lax
from jax.experimental import pallas as pl
from jax.experimental.pallas import tpu as pltpu
from jax.experimental.pallas import tpu_sc as plsc

N = 10000
E = 320000
H = 128
C = 10
L = 3
G = 128
BN_EPS = 1e-5

NC = 2
NS = 16
NW = NC * NS

EB = 128
K_TILE = 80
E_PAD = NW * K_TILE * EB
GRP = 8
NGRP = K_TILE // GRP

N_PAD = 10240
RPT = N_PAD // NS
BLK = 1024
NBLK = N_PAD // BLK


def _sc_agg(h_pad, zeros_pad, edges3d):
    mesh = plsc.VectorSubcoreMesh(
        core_axis_name="c", subcore_axis_name="s", num_cores=NC, num_subcores=NS
    )

    @functools.partial(
        pl.kernel,
        out_type=jax.ShapeDtypeStruct((NC, N_PAD, H), jnp.float32),
        mesh=mesh,
        scratch_types=[
            pltpu.VMEM((K_TILE, EB), jnp.int32),
            pltpu.VMEM((2, GRP, EB), jnp.int32),
            pltpu.VMEM((EB, H), jnp.float32),
            pltpu.VMEM((EB, H), jnp.float32),
            pltpu.VMEM_SHARED((N_PAD, H), jnp.float32),
            pltpu.SemaphoreType.DMA,
            pltpu.SemaphoreType.DMA,
            pltpu.SemaphoreType.DMA,
        ],
    )
    def agg(h_hbm, zero_hbm, e_hbm, out_hbm, src_v, dst_i, rows_a,
            rows_b, acc, sem_a, sem_b, sem_i):
        c = lax.axis_index("c")
        s = lax.axis_index("s")
        wid = s * NC + c
        @pl.when(c == 0)
        def _():
            pltpu.async_copy(
                h_hbm.at[pl.ds(s * RPT, RPT)], acc.at[pl.ds(s * RPT, RPT)], sem_i
            )

        @pl.when(c == 1)
        def _():
            pltpu.async_copy(
                zero_hbm.at[pl.ds(s * RPT, RPT)], acc.at[pl.ds(s * RPT, RPT)], sem_i
            )

        init = pltpu.make_async_copy(
            h_hbm.at[pl.ds(s * RPT, RPT)], acc.at[pl.ds(s * RPT, RPT)], sem_i
        )
        base = wid * K_TILE
        pltpu.sync_copy(e_hbm.at[0, pl.ds(base, K_TILE)], src_v)
        pltpu.sync_copy(e_hbm.at[1, pl.ds(base, GRP)], dst_i.at[0])
        pltpu.async_copy(h_hbm.at[src_v.at[0]], rows_a, sem_a)
        init.wait()
        plsc.subcore_barrier()


        def group(g, carry):
            slot = lax.rem(g, 2)

            @pl.when(g + 1 < NGRP)
            def _():
                pltpu.async_copy(
                    e_hbm.at[1, pl.ds(base + (g + 1) * GRP, GRP)],
                    dst_i.at[lax.rem(g + 1, 2)],
                    sem_i,
                )

            for p in range(GRP // 2):
                j0 = g * GRP + 2 * p
                pltpu.async_copy(h_hbm.at[src_v.at[j0 + 1]], rows_b, sem_b)
                pltpu.make_async_copy(h_hbm.at[src_v.at[j0]], rows_a, sem_a).wait()
                pltpu.sync_copy(rows_a, acc.at[dst_i.at[slot, 2 * p]], add=True)
                if p + 1 < GRP // 2:
                    pltpu.async_copy(h_hbm.at[src_v.at[j0 + 2]], rows_a, sem_a)
                else:

                    @pl.when(g + 1 < NGRP)
                    def _():
                        pltpu.async_copy(h_hbm.at[src_v.at[j0 + 2]], rows_a, sem_a)

                pltpu.make_async_copy(h_hbm.at[src_v.at[j0 + 1]], rows_b, sem_b).wait()
                pltpu.sync_copy(rows_b, acc.at[dst_i.at[slot, 2 * p + 1]], add=True)

            @pl.when(g + 1 < NGRP)
            def _():
                pltpu.make_async_copy(
                    e_hbm.at[1, pl.ds(base + (g + 1) * GRP, GRP)],
                    dst_i.at[lax.rem(g + 1, 2)],
                    sem_i,
                ).wait()

            return carry

        lax.fori_loop(0, NGRP, group, 0)
        plsc.subcore_barrier()
        pltpu.sync_copy(
            acc.at[pl.ds(s * RPT, RPT)], out_hbm.at[c, pl.ds(s * RPT, RPT)]
        )

    return agg(h_pad, zeros_pad, edges3d)


def _mlp_body(a_ref, b_ref, w1_ref, bb_ref, w2_ref, b2_ref, o_ref):
    z = a_ref[0] + b_ref[0]
    t = jnp.dot(z, w1_ref[...], preferred_element_type=jnp.float32) + bb_ref[0:1, :]
    t = jnp.maximum(t, 0.0)
    u = jnp.dot(t, w2_ref[...], preferred_element_type=jnp.float32) + b2_ref[0:1, :]
    o_ref[...] = jnp.maximum(u, 0.0)


def _mlp(acc, w1f, bb, w2, b2):
    row = lambda i: (i, 0)
    fixed = lambda i: (0, 0)
    return pl.pallas_call(
        _mlp_body,
        grid=(NBLK,),
        in_specs=[
            pl.BlockSpec((1, BLK, H), lambda i: (0, i, 0)),
            pl.BlockSpec((1, BLK, H), lambda i: (1, i, 0)),
            pl.BlockSpec((H, H), fixed),
            pl.BlockSpec((8, H), fixed),
            pl.BlockSpec((H, H), fixed),
            pl.BlockSpec((8, H), fixed),
        ],
        out_specs=pl.BlockSpec((BLK, H), row),
        out_shape=jax.ShapeDtypeStruct((N_PAD, H), jnp.float32),
    )(acc, acc, w1f, bb, w2, b2)


def _mlp_pool_body(
    a_ref, b_ref, w1_ref, bb_ref, w2_ref, b2_ref,
    batch_ref, wout_ref, bout_ref, o_ref, pool_acc,
):
    i = pl.program_id(0)
    z = a_ref[0] + b_ref[0]
    t = jnp.dot(z, w1_ref[...], preferred_element_type=jnp.float32) + bb_ref[0:1, :]
    t = jnp.maximum(t, 0.0)
    u = jnp.dot(t, w2_ref[...], preferred_element_type=jnp.float32) + b2_ref[0:1, :]
    hb = jnp.maximum(u, 0.0)

    bi = batch_ref[0, 0, :]
    seg = lax.broadcasted_iota(jnp.int32, (G, BLK), 0)
    m = (bi[None, :] == seg).astype(jnp.float32)
    part = jnp.dot(m, hb, preferred_element_type=jnp.float32)

    @pl.when(i == 0)
    def _():
        pool_acc[...] = jnp.zeros_like(pool_acc)

    pool_acc[...] += part

    @pl.when(i == NBLK - 1)
    def _():
        o_ref[...] = (
            jnp.dot(pool_acc[...], wout_ref[...], preferred_element_type=jnp.float32)
            + bout_ref[0:1, :]
        )


def _mlp_pool(acc, w1f, bb, w2, b2, batch3d, wout_p, bout_p):
    fixed = lambda i: (0, 0)
    return pl.pallas_call(
        _mlp_pool_body,
        grid=(NBLK,),
        in_specs=[
            pl.BlockSpec((1, BLK, H), lambda i: (0, i, 0)),
            pl.BlockSpec((1, BLK, H), lambda i: (1, i, 0)),
            pl.BlockSpec((H, H), fixed),
            pl.BlockSpec((8, H), fixed),
            pl.BlockSpec((H, H), fixed),
            pl.BlockSpec((8, H), fixed),
            pl.BlockSpec((1, 1, BLK), lambda i: (i, 0, 0)),
            pl.BlockSpec((H, H), fixed),
            pl.BlockSpec((8, H), fixed),
        ],
        out_specs=pl.BlockSpec((G, H), fixed),
        out_shape=jax.ShapeDtypeStruct((G, H), jnp.float32),
        scratch_shapes=[pltpu.VMEM((G, H), jnp.float32)],
    )(acc, acc, w1f, bb, w2, b2, batch3d, wout_p, bout_p)


def kernel(x, edge_index, batch, params):
    h = jnp.zeros((N_PAD, H), jnp.float32).at[:N].set(x)

    pad_rows = NW * K_TILE - E // EB
    fill = N + (jnp.arange(pad_rows * EB, dtype=jnp.int32) % (N_PAD - N))
    fill3 = jnp.broadcast_to(
        fill.reshape(pad_rows, EB)[None], (2, pad_rows, EB)
    ).astype(jnp.int32)
    edges3d = jnp.concatenate(
        [edge_index.reshape(2, E // EB, EB), fill3], axis=1
    )

    inv_std = 1.0 / jnp.sqrt(1.0 + BN_EPS)
    bc8 = lambda v: jnp.broadcast_to(v[None, :], (8, H))

    batch3d = (
        jnp.concatenate([batch, jnp.full((N_PAD - N,), G, jnp.int32)])
        .reshape(NBLK, 1, BLK)
    )
    wout_p = jnp.zeros((H, H), jnp.float32).at[:, :C].set(params["Wout"])
    bout_p = bc8(jnp.zeros((H,), jnp.float32).at[:C].set(params["bout"]))

    zeros_pad = jnp.zeros((N_PAD, H), jnp.float32)
    out = None
    for i in range(L):
        sc = inv_std * params[f"gamma_{i}"]
        w1f = params[f"W1_{i}"] * sc[None, :]
        bb = bc8(params[f"b1_{i}"] * sc + params[f"beta_{i}"])
        b2 = bc8(params[f"b2_{i}"])
        acc = _sc_agg(h, zeros_pad, edges3d)
        if i < L - 1:
            h = _mlp(acc, w1f, bb, params[f"W2_{i}"], b2)
        else:
            out = _mlp_pool(
                acc, w1f, bb, params[f"W2_{i}"], b2,
                batch3d, wout_p, bout_p,
            )
    return out[:, :C]
